# baseline (speedup 1.0000x reference)
import functools

import jax
import jax.numpy as jnp
from jax import lax
from jax.experimental import pallas as pl
from jax.experimental.pallas import tpu as pltpu
from jax.experimental.pallas import tpu_sc as plsc

B, K, N, DIM, C_IN = 4, 1024, 2048, 3, 125
IN_SIZE = C_IN + DIM
H1 = H2 = C_OUT = 256
NB = 16

NC, NS = 2, 16
NW = NC * NS
NQ = B * K
QPW = NQ // NW
TPB = NW // B
NCHUNK = N // 16


def _topk_body(pts_hbm, keys_hbm, out_hbm, pts_v, keys_v, outbuf, dbuf):
    c = lax.axis_index("c")
    s = lax.axis_index("s")
    wid = s * NC + c
    b = wid // TPB
    pltpu.sync_copy(pts_hbm.at[b], pts_v)
    pltpu.sync_copy(keys_hbm.at[wid], keys_v)

    def gbody(gi, _):
        g0 = gi * 16
        kxv = keys_v[pl.ds(g0, 16)]
        kyv = keys_v[pl.ds(QPW + g0, 16)]
        kzv = keys_v[pl.ds(2 * QPW + g0, 16)]
        for j in range(16):
            kx = kxv[j]
            ky = kyv[j]
            kz = kzv[j]

            def c1(ci, m):
                off = ci * 16
                px = pts_v[pl.ds(off, 16)]
                py = pts_v[pl.ds(N + off, 16)]
                pz = pts_v[pl.ds(2 * N + off, 16)]
                dx = px - kx
                dy = py - ky
                dz = pz - kz
                d = dx * dx + dy * dy + dz * dz
                dbuf[pl.ds(off, 16)] = d
                return jnp.minimum(m, d)

            m = lax.fori_loop(0, NCHUNK, c1,
                              jnp.full((16,), jnp.inf, jnp.float32))
            t = jnp.max(m)

            def c2(ci, carry):
                off = ci * 16
                d = dbuf[pl.ds(off, 16)]
                hit = jnp.any(d <= t)

                def merge(c):
                    rd, ri = c
                    idxv = off + lax.iota(jnp.int32, 16)
                    ds_, is_ = plsc.sort_key_val(d, idxv)
                    rev_d = lax.rev(ds_, (0,))
                    rev_i = lax.rev(is_, (0,))
                    take = rev_d < rd
                    md = jnp.where(take, rev_d, rd)
                    mi = jnp.where(take, rev_i, ri)
                    return tuple(plsc.sort_key_val(md, mi))

                return lax.cond(hit, merge, lambda c: c, carry)

            init = (jnp.full((16,), jnp.inf, jnp.float32),
                    jnp.zeros((16,), jnp.int32))
            _, ri = lax.fori_loop(0, NCHUNK, c2, init)
            outbuf[pl.ds((g0 + j) * NB, NB)] = ri + b * N
        return 0

    lax.fori_loop(0, QPW // 16, gbody, 0)
    pltpu.sync_copy(outbuf, out_hbm.at[pl.ds(wid * QPW * NB, QPW * NB)])


def _topk_call(pts_t, keys_g):
    mesh = plsc.VectorSubcoreMesh(core_axis_name="c", subcore_axis_name="s",
                                  num_cores=NC, num_subcores=NS)
    fn = functools.partial(
        pl.kernel,
        out_type=jax.ShapeDtypeStruct((NQ * NB,), jnp.int32),
        mesh=mesh,
        compiler_params=pltpu.CompilerParams(needs_layout_passes=False),
        scratch_types=[
            pltpu.VMEM((DIM * N,), jnp.float32),
            pltpu.VMEM((DIM * QPW,), jnp.float32),
            pltpu.VMEM((QPW * NB,), jnp.int32),
            pltpu.VMEM((N,), jnp.float32),
        ],
    )(_topk_body)
    return fn(pts_t, keys_g).reshape(NQ, NB)


GROWS = NQ * NB
RPW = GROWS // NW
GCH = 128
NGC = RPW // GCH


def _gather_body(table_hbm, gidx_hbm, out_hbm, idx_v, rows_v, sem):
    c = lax.axis_index("c")
    s = lax.axis_index("s")
    wid = s * NC + c
    base = wid * RPW

    def body(t, _):
        r0 = base + t * GCH
        pltpu.sync_copy(gidx_hbm.at[pl.ds(r0, GCH)], idx_v)
        pltpu.async_copy(table_hbm.at[idx_v], rows_v, sem).wait()
        pltpu.sync_copy(rows_v, out_hbm.at[pl.ds(r0, GCH)])
        return 0

    lax.fori_loop(0, NGC, body, 0)


def _gather_call(table, gidx):
    mesh = plsc.VectorSubcoreMesh(core_axis_name="c", subcore_axis_name="s",
                                  num_cores=NC, num_subcores=NS)
    fn = functools.partial(
        pl.kernel,
        out_type=jax.ShapeDtypeStruct((GROWS, IN_SIZE), jnp.float32),
        mesh=mesh,
        scratch_types=[
            pltpu.VMEM((GCH,), jnp.int32),
            pltpu.VMEM((GCH, IN_SIZE), jnp.float32),
            pltpu.SemaphoreType.DMA,
        ],
    )(_gather_body)
    return fn(table, gidx)


QB = 256


def _mlp_body(g_ref, kn_ref, w1_ref, w1k_ref, b1_ref, w2_ref, b2_ref,
              w3_ref, b3_ref, out_ref):
    f32 = jnp.float32
    kt = jnp.dot(kn_ref[...], w1k_ref[...], preferred_element_type=f32)
    kt = kt + b1_ref[...]
    acc = jnp.full((QB, C_OUT), -jnp.inf, f32)
    for j in range(NB):
        x = g_ref[j]
        h = jnp.dot(x, w1_ref[...], preferred_element_type=f32) + kt
        h = jnp.maximum(h, 0.0)
        h = jnp.dot(h, w2_ref[...], preferred_element_type=f32) + b2_ref[...]
        h = jnp.maximum(h, 0.0)
        o = jnp.dot(h, w3_ref[...], preferred_element_type=f32)
        acc = jnp.maximum(acc, o)
    out_ref[...] = acc + b3_ref[...]


def _mlp_call(g, kn, W1, W1k, b1, W2, b2, W3, b3):
    grid = (NQ // QB,)
    return pl.pallas_call(
        _mlp_body,
        grid=grid,
        in_specs=[
            pl.BlockSpec((NB, QB, IN_SIZE), lambda i: (0, i, 0)),
            pl.BlockSpec((QB, 8), lambda i: (i, 0)),
            pl.BlockSpec((IN_SIZE, H1), lambda i: (0, 0)),
            pl.BlockSpec((8, H1), lambda i: (0, 0)),
            pl.BlockSpec((1, H1), lambda i: (0, 0)),
            pl.BlockSpec((H1, H2), lambda i: (0, 0)),
            pl.BlockSpec((1, H2), lambda i: (0, 0)),
            pl.BlockSpec((H2, C_OUT), lambda i: (0, 0)),
            pl.BlockSpec((1, C_OUT), lambda i: (0, 0)),
        ],
        out_specs=pl.BlockSpec((QB, C_OUT), lambda i: (i, 0)),
        out_shape=jax.ShapeDtypeStruct((NQ, C_OUT), jnp.float32),
    )(g, kn, W1, W1k, b1, W2, b2, W3, b3)


def kernel(keys, points, feats, W1, b1, W2, b2, W3, b3):
    f32 = jnp.float32
    pts_t = points.transpose(0, 2, 1).reshape(B, DIM * N)
    keys_g = (keys.reshape(B, TPB, QPW, DIM)
              .transpose(0, 1, 3, 2)
              .reshape(NW, DIM * QPW))

    idx = _topk_call(pts_t, keys_g)
    gidx = idx.T.reshape(-1)

    table = jnp.concatenate([points, feats], axis=2).reshape(B * N, IN_SIZE)
    g = _gather_call(table, gidx).reshape(NB, NQ, IN_SIZE)

    kflat = keys.reshape(NQ, DIM)
    kn = jnp.concatenate([-kflat, jnp.zeros((NQ, 8 - DIM), f32)], axis=1)
    W1k = jnp.concatenate([W1[:DIM], jnp.zeros((8 - DIM, H1), f32)], axis=0)

    out = _mlp_call(g, kn, W1, W1k, b1.reshape(1, H1), W2, b2.reshape(1, H2),
                    W3, b3.reshape(1, C_OUT))
    return out.reshape(B, K, C_OUT)


# speedup vs baseline: 2.1388x; 2.1388x over previous
import functools

import jax
import jax.numpy as jnp
from jax import lax
from jax.experimental import pallas as pl
from jax.experimental.pallas import tpu as pltpu
from jax.experimental.pallas import tpu_sc as plsc

B, K, N, DIM, C_IN = 4, 1024, 2048, 3, 125
IN_SIZE = C_IN + DIM
H1 = H2 = C_OUT = 256
NB = 16

NC, NS = 2, 16
NW = NC * NS
NQ = B * K
QPW = NQ // NW
TPB = NW // B
NCHUNK = N // 16


def _topk_body(pts_hbm, keys_hbm, out_hbm, pts_v, keys_v, outbuf, dbuf):
    c = lax.axis_index("c")
    s = lax.axis_index("s")
    wid = s * NC + c
    b = wid // TPB
    pltpu.sync_copy(pts_hbm.at[b], pts_v)
    pltpu.sync_copy(keys_hbm.at[wid], keys_v)

    def gbody(gi, _):
        g0 = gi * 16
        kxv = keys_v[pl.ds(g0, 16)]
        kyv = keys_v[pl.ds(QPW + g0, 16)]
        kzv = keys_v[pl.ds(2 * QPW + g0, 16)]
        for j in range(0, 16, 2):
            kx0, ky0, kz0 = kxv[j], kyv[j], kzv[j]
            kx1, ky1, kz1 = kxv[j + 1], kyv[j + 1], kzv[j + 1]

            def cbody(ci, carry):
                rd0, ri0, rd1, ri1 = carry
                off = ci * 16
                px = pts_v[pl.ds(off, 16)]
                py = pts_v[pl.ds(N + off, 16)]
                pz = pts_v[pl.ds(2 * N + off, 16)]
                idxv = off + lax.iota(jnp.int32, 16)

                dx = px - kx0
                dy = py - ky0
                dz = pz - kz0
                d0 = dx * dx + dy * dy + dz * dz
                sk0, sv0 = plsc.sort_key_val(d0, idxv, descending=True)
                take0 = sk0 < rd0
                md0 = jnp.where(take0, sk0, rd0)
                mi0 = jnp.where(take0, sv0, ri0)
                rd0, ri0 = plsc.sort_key_val(md0, mi0)

                dx = px - kx1
                dy = py - ky1
                dz = pz - kz1
                d1 = dx * dx + dy * dy + dz * dz
                sk1, sv1 = plsc.sort_key_val(d1, idxv, descending=True)
                take1 = sk1 < rd1
                md1 = jnp.where(take1, sk1, rd1)
                mi1 = jnp.where(take1, sv1, ri1)
                rd1, ri1 = plsc.sort_key_val(md1, mi1)
                return (rd0, ri0, rd1, ri1)

            inf16 = jnp.full((16,), jnp.inf, jnp.float32)
            z16 = jnp.zeros((16,), jnp.int32)
            _, ri0, _, ri1 = lax.fori_loop(0, NCHUNK, cbody,
                                           (inf16, z16, inf16, z16),
                                           unroll=2)
            outbuf[pl.ds((g0 + j) * NB, NB)] = ri0 + b * N
            outbuf[pl.ds((g0 + j + 1) * NB, NB)] = ri1 + b * N
        return 0

    lax.fori_loop(0, QPW // 16, gbody, 0)
    pltpu.sync_copy(outbuf, out_hbm.at[pl.ds(wid * QPW * NB, QPW * NB)])


def _topk_call(pts_t, keys_g):
    mesh = plsc.VectorSubcoreMesh(core_axis_name="c", subcore_axis_name="s",
                                  num_cores=NC, num_subcores=NS)
    fn = functools.partial(
        pl.kernel,
        out_type=jax.ShapeDtypeStruct((NQ * NB,), jnp.int32),
        mesh=mesh,
        compiler_params=pltpu.CompilerParams(needs_layout_passes=False),
        scratch_types=[
            pltpu.VMEM((DIM * N,), jnp.float32),
            pltpu.VMEM((DIM * QPW,), jnp.float32),
            pltpu.VMEM((QPW * NB,), jnp.int32),
            pltpu.VMEM((N,), jnp.float32),
        ],
    )(_topk_body)
    return fn(pts_t, keys_g).reshape(NQ, NB)


GROWS = NQ * NB
RPW = GROWS // NW
GCH = 128
NGC = RPW // GCH


def _gather_body(table_hbm, gidx_hbm, out_hbm, idx_v, rows_v, sem):
    c = lax.axis_index("c")
    s = lax.axis_index("s")
    wid = s * NC + c
    base = wid * RPW

    def body(t, _):
        r0 = base + t * GCH
        pltpu.sync_copy(gidx_hbm.at[pl.ds(r0, GCH)], idx_v)
        pltpu.async_copy(table_hbm.at[idx_v], rows_v, sem).wait()
        pltpu.sync_copy(rows_v, out_hbm.at[pl.ds(r0, GCH)])
        return 0

    lax.fori_loop(0, NGC, body, 0)


def _gather_call(table, gidx):
    mesh = plsc.VectorSubcoreMesh(core_axis_name="c", subcore_axis_name="s",
                                  num_cores=NC, num_subcores=NS)
    fn = functools.partial(
        pl.kernel,
        out_type=jax.ShapeDtypeStruct((GROWS, IN_SIZE), jnp.float32),
        mesh=mesh,
        scratch_types=[
            pltpu.VMEM((GCH,), jnp.int32),
            pltpu.VMEM((GCH, IN_SIZE), jnp.float32),
            pltpu.SemaphoreType.DMA,
        ],
    )(_gather_body)
    return fn(table, gidx)


QB = 256


def _mlp_body(g_ref, kn_ref, w1_ref, w1k_ref, b1_ref, w2_ref, b2_ref,
              w3_ref, b3_ref, out_ref):
    f32 = jnp.float32
    kt = jnp.dot(kn_ref[...], w1k_ref[...], preferred_element_type=f32)
    kt = kt + b1_ref[...]
    acc = jnp.full((QB, C_OUT), -jnp.inf, f32)
    for j in range(NB):
        x = g_ref[j]
        h = jnp.dot(x, w1_ref[...], preferred_element_type=f32) + kt
        h = jnp.maximum(h, 0.0)
        h = jnp.dot(h, w2_ref[...], preferred_element_type=f32) + b2_ref[...]
        h = jnp.maximum(h, 0.0)
        o = jnp.dot(h, w3_ref[...], preferred_element_type=f32)
        acc = jnp.maximum(acc, o)
    out_ref[...] = acc + b3_ref[...]


def _mlp_call(g, kn, W1, W1k, b1, W2, b2, W3, b3):
    grid = (NQ // QB,)
    return pl.pallas_call(
        _mlp_body,
        grid=grid,
        in_specs=[
            pl.BlockSpec((NB, QB, IN_SIZE), lambda i: (0, i, 0)),
            pl.BlockSpec((QB, 8), lambda i: (i, 0)),
            pl.BlockSpec((IN_SIZE, H1), lambda i: (0, 0)),
            pl.BlockSpec((8, H1), lambda i: (0, 0)),
            pl.BlockSpec((1, H1), lambda i: (0, 0)),
            pl.BlockSpec((H1, H2), lambda i: (0, 0)),
            pl.BlockSpec((1, H2), lambda i: (0, 0)),
            pl.BlockSpec((H2, C_OUT), lambda i: (0, 0)),
            pl.BlockSpec((1, C_OUT), lambda i: (0, 0)),
        ],
        out_specs=pl.BlockSpec((QB, C_OUT), lambda i: (i, 0)),
        out_shape=jax.ShapeDtypeStruct((NQ, C_OUT), jnp.float32),
    )(g, kn, W1, W1k, b1, W2, b2, W3, b3)


def kernel(keys, points, feats, W1, b1, W2, b2, W3, b3):
    f32 = jnp.float32
    pts_t = points.transpose(0, 2, 1).reshape(B, DIM * N)
    keys_g = (keys.reshape(B, TPB, QPW, DIM)
              .transpose(0, 1, 3, 2)
              .reshape(NW, DIM * QPW))

    idx = _topk_call(pts_t, keys_g)
    gidx = idx.T.reshape(-1)

    table = jnp.concatenate([points, feats], axis=2).reshape(B * N, IN_SIZE)
    g = _gather_call(table, gidx).reshape(NB, NQ, IN_SIZE)

    kflat = keys.reshape(NQ, DIM)
    kn = jnp.concatenate([-kflat, jnp.zeros((NQ, 8 - DIM), f32)], axis=1)
    W1k = jnp.concatenate([W1[:DIM], jnp.zeros((8 - DIM, H1), f32)], axis=0)

    out = _mlp_call(g, kn, W1, W1k, b1.reshape(1, H1), W2, b2.reshape(1, H2),
                    W3, b3.reshape(1, C_OUT))
    return out.reshape(B, K, C_OUT)


# speedup vs baseline: 2.2937x; 1.0724x over previous
import functools

import jax
import jax.numpy as jnp
from jax import lax
from jax.experimental import pallas as pl
from jax.experimental.pallas import tpu as pltpu
from jax.experimental.pallas import tpu_sc as plsc

B, K, N, DIM, C_IN = 4, 1024, 2048, 3, 125
IN_SIZE = C_IN + DIM
H1 = H2 = C_OUT = 256
NB = 16

NC, NS = 2, 16
NW = NC * NS
NQ = B * K
QPW = NQ // NW
TPB = NW // B
NCHUNK = N // 16


def _topk_body(pts_hbm, keys_hbm, out_hbm, pts_v, keys_v, outbuf, dbuf):
    c = lax.axis_index("c")
    s = lax.axis_index("s")
    wid = s * NC + c
    b = wid // TPB
    pltpu.sync_copy(pts_hbm.at[b], pts_v)
    pltpu.sync_copy(keys_hbm.at[wid], keys_v)

    def gbody(gi, _):
        g0 = gi * 16
        kxv = keys_v[pl.ds(g0, 16)]
        kyv = keys_v[pl.ds(QPW + g0, 16)]
        kzv = keys_v[pl.ds(2 * QPW + g0, 16)]
        for j in range(0, 16, 2):
            kx0, ky0, kz0 = kxv[j], kyv[j], kzv[j]
            kx1, ky1, kz1 = kxv[j + 1], kyv[j + 1], kzv[j + 1]

            def cbody(ci, carry):
                rd0, ri0, rd1, ri1 = carry
                off = ci * 16
                px = pts_v[pl.ds(off, 16)]
                py = pts_v[pl.ds(N + off, 16)]
                pz = pts_v[pl.ds(2 * N + off, 16)]
                idxv = off + lax.iota(jnp.int32, 16)

                dx = px - kx0
                dy = py - ky0
                dz = pz - kz0
                d0 = dx * dx + dy * dy + dz * dz
                sk0, sv0 = plsc.sort_key_val(d0, idxv, descending=True)
                take0 = sk0 < rd0
                md0 = jnp.where(take0, sk0, rd0)
                mi0 = jnp.where(take0, sv0, ri0)
                rd0, ri0 = plsc.sort_key_val(md0, mi0)

                dx = px - kx1
                dy = py - ky1
                dz = pz - kz1
                d1 = dx * dx + dy * dy + dz * dz
                sk1, sv1 = plsc.sort_key_val(d1, idxv, descending=True)
                take1 = sk1 < rd1
                md1 = jnp.where(take1, sk1, rd1)
                mi1 = jnp.where(take1, sv1, ri1)
                rd1, ri1 = plsc.sort_key_val(md1, mi1)
                return (rd0, ri0, rd1, ri1)

            inf16 = jnp.full((16,), jnp.inf, jnp.float32)
            z16 = jnp.zeros((16,), jnp.int32)
            _, ri0, _, ri1 = lax.fori_loop(0, NCHUNK, cbody,
                                           (inf16, z16, inf16, z16),
                                           unroll=2)
            outbuf[pl.ds((g0 + j) * NB, NB)] = ri0 + b * N
            outbuf[pl.ds((g0 + j + 1) * NB, NB)] = ri1 + b * N
        return 0

    lax.fori_loop(0, QPW // 16, gbody, 0)
    pltpu.sync_copy(outbuf, out_hbm.at[pl.ds(wid * QPW * NB, QPW * NB)])


def _topk_call(pts_t, keys_g):
    mesh = plsc.VectorSubcoreMesh(core_axis_name="c", subcore_axis_name="s",
                                  num_cores=NC, num_subcores=NS)
    fn = functools.partial(
        pl.kernel,
        out_type=jax.ShapeDtypeStruct((NQ * NB,), jnp.int32),
        mesh=mesh,
        compiler_params=pltpu.CompilerParams(needs_layout_passes=False),
        scratch_types=[
            pltpu.VMEM((DIM * N,), jnp.float32),
            pltpu.VMEM((DIM * QPW,), jnp.float32),
            pltpu.VMEM((QPW * NB,), jnp.int32),
            pltpu.VMEM((N,), jnp.float32),
        ],
    )(_topk_body)
    return fn(pts_t, keys_g).reshape(NQ, NB)


GROWS = NQ * NB
RPW = GROWS // NW
GCH = 128
NGC = RPW // GCH


def _gather_body(table_hbm, gidx_hbm, out_hbm, idx_all, rows_v0, rows_v1,
                 sem0, sem1):
    c = lax.axis_index("c")
    s = lax.axis_index("s")
    wid = s * NC + c
    base = wid * RPW
    pltpu.sync_copy(gidx_hbm.at[pl.ds(base, RPW)], idx_all)

    def start(t, rows, sem):
        idx = idx_all.at[pl.ds(t * GCH, GCH)]
        pltpu.async_copy(table_hbm.at[idx], rows, sem)

    def drain(rows, sem):
        pltpu.make_async_copy(table_hbm.at[pl.ds(0, GCH)], rows, sem).wait()

    start(0, rows_v0, sem0)

    def body(i, _):
        ta = 2 * i
        tb = 2 * i + 1
        tc_ = lax.rem(2 * i + 2, NGC)
        start(tb, rows_v1, sem1)
        drain(rows_v0, sem0)
        pltpu.sync_copy(rows_v0, out_hbm.at[pl.ds(base + ta * GCH, GCH)])
        start(tc_, rows_v0, sem0)
        drain(rows_v1, sem1)
        pltpu.sync_copy(rows_v1, out_hbm.at[pl.ds(base + tb * GCH, GCH)])
        return 0

    lax.fori_loop(0, NGC // 2, body, 0)
    drain(rows_v0, sem0)
    pltpu.sync_copy(rows_v0, out_hbm.at[pl.ds(base, GCH)])


def _gather_call(table, gidx):
    mesh = plsc.VectorSubcoreMesh(core_axis_name="c", subcore_axis_name="s",
                                  num_cores=NC, num_subcores=NS)
    fn = functools.partial(
        pl.kernel,
        out_type=jax.ShapeDtypeStruct((GROWS, IN_SIZE), jnp.float32),
        mesh=mesh,
        scratch_types=[
            pltpu.VMEM((RPW,), jnp.int32),
            pltpu.VMEM((GCH, IN_SIZE), jnp.float32),
            pltpu.VMEM((GCH, IN_SIZE), jnp.float32),
            pltpu.SemaphoreType.DMA,
            pltpu.SemaphoreType.DMA,
        ],
    )(_gather_body)
    return fn(table, gidx)


QB = 256


def _mlp_body(g_ref, kn_ref, w1_ref, w1k_ref, b1_ref, w2_ref, b2_ref,
              w3_ref, b3_ref, out_ref):
    f32 = jnp.float32
    kt = jnp.dot(kn_ref[...], w1k_ref[...], preferred_element_type=f32)
    kt = kt + b1_ref[...]
    acc = jnp.full((QB, C_OUT), -jnp.inf, f32)
    for j in range(NB):
        x = g_ref[j]
        h = jnp.dot(x, w1_ref[...], preferred_element_type=f32) + kt
        h = jnp.maximum(h, 0.0)
        h = jnp.dot(h, w2_ref[...], preferred_element_type=f32) + b2_ref[...]
        h = jnp.maximum(h, 0.0)
        o = jnp.dot(h, w3_ref[...], preferred_element_type=f32)
        acc = jnp.maximum(acc, o)
    out_ref[...] = acc + b3_ref[...]


def _mlp_call(g, kn, W1, W1k, b1, W2, b2, W3, b3):
    grid = (NQ // QB,)
    return pl.pallas_call(
        _mlp_body,
        grid=grid,
        in_specs=[
            pl.BlockSpec((NB, QB, IN_SIZE), lambda i: (0, i, 0)),
            pl.BlockSpec((QB, 8), lambda i: (i, 0)),
            pl.BlockSpec((IN_SIZE, H1), lambda i: (0, 0)),
            pl.BlockSpec((8, H1), lambda i: (0, 0)),
            pl.BlockSpec((1, H1), lambda i: (0, 0)),
            pl.BlockSpec((H1, H2), lambda i: (0, 0)),
            pl.BlockSpec((1, H2), lambda i: (0, 0)),
            pl.BlockSpec((H2, C_OUT), lambda i: (0, 0)),
            pl.BlockSpec((1, C_OUT), lambda i: (0, 0)),
        ],
        out_specs=pl.BlockSpec((QB, C_OUT), lambda i: (i, 0)),
        out_shape=jax.ShapeDtypeStruct((NQ, C_OUT), jnp.float32),
    )(g, kn, W1, W1k, b1, W2, b2, W3, b3)


def kernel(keys, points, feats, W1, b1, W2, b2, W3, b3):
    f32 = jnp.float32
    pts_t = points.transpose(0, 2, 1).reshape(B, DIM * N)
    keys_g = (keys.reshape(B, TPB, QPW, DIM)
              .transpose(0, 1, 3, 2)
              .reshape(NW, DIM * QPW))

    idx = _topk_call(pts_t, keys_g)
    gidx = idx.T.reshape(-1)

    table = jnp.concatenate([points, feats], axis=2).reshape(B * N, IN_SIZE)
    g = _gather_call(table, gidx).reshape(NB, NQ, IN_SIZE)

    kflat = keys.reshape(NQ, DIM)
    kn = jnp.concatenate([-kflat, jnp.zeros((NQ, 8 - DIM), f32)], axis=1)
    W1k = jnp.concatenate([W1[:DIM], jnp.zeros((8 - DIM, H1), f32)], axis=0)

    out = _mlp_call(g, kn, W1, W1k, b1.reshape(1, H1), W2, b2.reshape(1, H2),
                    W3, b3.reshape(1, C_OUT))
    return out.reshape(B, K, C_OUT)


# speedup vs baseline: 2.8321x; 1.2347x over previous
import functools

import jax
import jax.numpy as jnp
from jax import lax
from jax.experimental import pallas as pl
from jax.experimental.pallas import tpu as pltpu
from jax.experimental.pallas import tpu_sc as plsc

B, K, N, DIM, C_IN = 4, 1024, 2048, 3, 125
IN_SIZE = C_IN + DIM
H1 = H2 = C_OUT = 256
NB = 16

NC, NS = 2, 16
NW = NC * NS
NQ = B * K
QPW = NQ // NW
TPB = NW // B
NCHUNK = N // 16
ILV = 4
UNROLL = 2


def _topk_body(pts_hbm, keys_hbm, out_hbm, pts_v, keys_v, outbuf, dbuf):
    c = lax.axis_index("c")
    s = lax.axis_index("s")
    wid = s * NC + c
    b = wid // TPB
    pltpu.sync_copy(pts_hbm.at[b], pts_v)
    pltpu.sync_copy(keys_hbm.at[wid], keys_v)

    def gbody(gi, _):
        g0 = gi * 16
        kxv = keys_v[pl.ds(g0, 16)]
        kyv = keys_v[pl.ds(QPW + g0, 16)]
        kzv = keys_v[pl.ds(2 * QPW + g0, 16)]
        for j0 in range(0, 16, ILV):
            kq = [(kxv[j0 + t], kyv[j0 + t], kzv[j0 + t])
                  for t in range(ILV)]

            def cbody(ci, carry):
                off = ci * 16
                px = pts_v[pl.ds(off, 16)]
                py = pts_v[pl.ds(N + off, 16)]
                pz = pts_v[pl.ds(2 * N + off, 16)]
                idxv = off + lax.iota(jnp.int32, 16)
                nxt = []
                for t in range(ILV):
                    kx, ky, kz = kq[t]
                    rd, ri = carry[2 * t], carry[2 * t + 1]
                    dx = px - kx
                    dy = py - ky
                    dz = pz - kz
                    d = dx * dx + dy * dy + dz * dz
                    sk, sv = plsc.sort_key_val(d, idxv, descending=True)
                    take = sk < rd
                    md = jnp.where(take, sk, rd)
                    mi = jnp.where(take, sv, ri)
                    nxt += list(plsc.sort_key_val(md, mi))
                return tuple(nxt)

            inf16 = jnp.full((16,), jnp.inf, jnp.float32)
            z16 = jnp.zeros((16,), jnp.int32)
            res = lax.fori_loop(0, NCHUNK, cbody, (inf16, z16) * ILV,
                                unroll=UNROLL)
            for t in range(ILV):
                outbuf[pl.ds((g0 + j0 + t) * NB, NB)] = res[2 * t + 1] + b * N
        return 0

    lax.fori_loop(0, QPW // 16, gbody, 0)
    pltpu.sync_copy(outbuf, out_hbm.at[pl.ds(wid * QPW * NB, QPW * NB)])


def _topk_call(pts_t, keys_g):
    mesh = plsc.VectorSubcoreMesh(core_axis_name="c", subcore_axis_name="s",
                                  num_cores=NC, num_subcores=NS)
    fn = functools.partial(
        pl.kernel,
        out_type=jax.ShapeDtypeStruct((NQ * NB,), jnp.int32),
        mesh=mesh,
        compiler_params=pltpu.CompilerParams(needs_layout_passes=False),
        scratch_types=[
            pltpu.VMEM((DIM * N,), jnp.float32),
            pltpu.VMEM((DIM * QPW,), jnp.float32),
            pltpu.VMEM((QPW * NB,), jnp.int32),
            pltpu.VMEM((N,), jnp.float32),
        ],
    )(_topk_body)
    return fn(pts_t, keys_g).reshape(NQ, NB)


GROWS = NQ * NB
RPW = GROWS // NW
GCH = 128
NGC = RPW // GCH


def _gather_body(table_hbm, gidx_hbm, out_hbm, idx_all, rows_v0, rows_v1,
                 sem0, sem1):
    c = lax.axis_index("c")
    s = lax.axis_index("s")
    wid = s * NC + c
    base = wid * RPW
    pltpu.sync_copy(gidx_hbm.at[pl.ds(base, RPW)], idx_all)

    def start(t, rows, sem):
        idx = idx_all.at[pl.ds(t * GCH, GCH)]
        pltpu.async_copy(table_hbm.at[idx], rows, sem)

    def drain(rows, sem):
        pltpu.make_async_copy(table_hbm.at[pl.ds(0, GCH)], rows, sem).wait()

    start(0, rows_v0, sem0)

    def body(i, _):
        ta = 2 * i
        tb = 2 * i + 1
        tc_ = lax.rem(2 * i + 2, NGC)
        start(tb, rows_v1, sem1)
        drain(rows_v0, sem0)
        pltpu.sync_copy(rows_v0, out_hbm.at[pl.ds(base + ta * GCH, GCH)])
        start(tc_, rows_v0, sem0)
        drain(rows_v1, sem1)
        pltpu.sync_copy(rows_v1, out_hbm.at[pl.ds(base + tb * GCH, GCH)])
        return 0

    lax.fori_loop(0, NGC // 2, body, 0)
    drain(rows_v0, sem0)
    pltpu.sync_copy(rows_v0, out_hbm.at[pl.ds(base, GCH)])


def _gather_call(table, gidx):
    mesh = plsc.VectorSubcoreMesh(core_axis_name="c", subcore_axis_name="s",
                                  num_cores=NC, num_subcores=NS)
    fn = functools.partial(
        pl.kernel,
        out_type=jax.ShapeDtypeStruct((GROWS, IN_SIZE), jnp.float32),
        mesh=mesh,
        scratch_types=[
            pltpu.VMEM((RPW,), jnp.int32),
            pltpu.VMEM((GCH, IN_SIZE), jnp.float32),
            pltpu.VMEM((GCH, IN_SIZE), jnp.float32),
            pltpu.SemaphoreType.DMA,
            pltpu.SemaphoreType.DMA,
        ],
    )(_gather_body)
    return fn(table, gidx)


QB = 256


def _mlp_body(g_ref, kn_ref, w1_ref, w1k_ref, b1_ref, w2_ref, b2_ref,
              w3_ref, b3_ref, out_ref):
    f32 = jnp.float32
    kt = jnp.dot(kn_ref[...], w1k_ref[...], preferred_element_type=f32)
    kt = kt + b1_ref[...]
    acc = jnp.full((QB, C_OUT), -jnp.inf, f32)
    for j in range(NB):
        x = g_ref[j]
        h = jnp.dot(x, w1_ref[...], preferred_element_type=f32) + kt
        h = jnp.maximum(h, 0.0)
        h = jnp.dot(h, w2_ref[...], preferred_element_type=f32) + b2_ref[...]
        h = jnp.maximum(h, 0.0)
        o = jnp.dot(h, w3_ref[...], preferred_element_type=f32)
        acc = jnp.maximum(acc, o)
    out_ref[...] = acc + b3_ref[...]


def _mlp_call(g, kn, W1, W1k, b1, W2, b2, W3, b3):
    grid = (NQ // QB,)
    return pl.pallas_call(
        _mlp_body,
        grid=grid,
        in_specs=[
            pl.BlockSpec((NB, QB, IN_SIZE), lambda i: (0, i, 0)),
            pl.BlockSpec((QB, 8), lambda i: (i, 0)),
            pl.BlockSpec((IN_SIZE, H1), lambda i: (0, 0)),
            pl.BlockSpec((8, H1), lambda i: (0, 0)),
            pl.BlockSpec((1, H1), lambda i: (0, 0)),
            pl.BlockSpec((H1, H2), lambda i: (0, 0)),
            pl.BlockSpec((1, H2), lambda i: (0, 0)),
            pl.BlockSpec((H2, C_OUT), lambda i: (0, 0)),
            pl.BlockSpec((1, C_OUT), lambda i: (0, 0)),
        ],
        out_specs=pl.BlockSpec((QB, C_OUT), lambda i: (i, 0)),
        out_shape=jax.ShapeDtypeStruct((NQ, C_OUT), jnp.float32),
    )(g, kn, W1, W1k, b1, W2, b2, W3, b3)


def kernel(keys, points, feats, W1, b1, W2, b2, W3, b3):
    f32 = jnp.float32
    pts_t = points.transpose(0, 2, 1).reshape(B, DIM * N)
    keys_g = (keys.reshape(B, TPB, QPW, DIM)
              .transpose(0, 1, 3, 2)
              .reshape(NW, DIM * QPW))

    idx = _topk_call(pts_t, keys_g)
    gidx = idx.T.reshape(-1)

    table = jnp.concatenate([points, feats], axis=2).reshape(B * N, IN_SIZE)
    g = _gather_call(table, gidx).reshape(NB, NQ, IN_SIZE)

    kflat = keys.reshape(NQ, DIM)
    kn = jnp.concatenate([-kflat, jnp.zeros((NQ, 8 - DIM), f32)], axis=1)
    W1k = jnp.concatenate([W1[:DIM], jnp.zeros((8 - DIM, H1), f32)], axis=0)

    out = _mlp_call(g, kn, W1, W1k, b1.reshape(1, H1), W2, b2.reshape(1, H2),
                    W3, b3.reshape(1, C_OUT))
    return out.reshape(B, K, C_OUT)
